# baseline (speedup 1.0000x reference)
import functools

import jax
import jax.numpy as jnp
from jax import lax
from jax.experimental import pallas as pl
from jax.experimental.pallas import tpu as pltpu
from jax.experimental.pallas import tpu_sc as plsc

N = 10000
D = 128
E = 320000
HALF = D // 2

NUM_SC = 2
NUM_TILES = 16
N_PAD = 10240
NPT = N_PAD // NUM_TILES
E_PAD = 327680
EROWS = E_PAD // 128
ROWS_MAIN = EROWS // NUM_TILES
ROWS_DEG = EROWS // (NUM_SC * NUM_TILES)

_mesh = plsc.VectorSubcoreMesh(core_axis_name="c", subcore_axis_name="s")


IROWS_DEG = 16


@functools.partial(
    pl.kernel,
    mesh=_mesh,
    out_type=jax.ShapeDtypeStruct((NUM_SC, N_PAD, 128), jnp.float32),
    scratch_types=[
        pltpu.VMEM((IROWS_DEG, 128), jnp.int32),
        pltpu.VMEM((128, 128), jnp.float32),
        pltpu.VMEM((64, 128), jnp.float32),
        pltpu.VMEM_SHARED((N_PAD, 128), jnp.float32),
    ],
)
def _deg_kernel(dst_hbm, ones_hbm, zeros_hbm, out_hbm, idx_v, ones_v, buf_v,
                acc_sh):
    c = lax.axis_index("c")
    s = lax.axis_index("s")
    wid = s * NUM_SC + c

    pltpu.sync_copy(ones_hbm, ones_v)
    pltpu.sync_copy(zeros_hbm, buf_v)

    def _zero(k, carry):
        pltpu.sync_copy(buf_v, acc_sh.at[pl.ds(s * NPT + k * 64, 64)])
        return carry

    lax.fori_loop(0, NPT // 64, _zero, 0)
    plsc.subcore_barrier()

    ibase = wid * ROWS_DEG

    def _chunk(k, carry):
        pltpu.sync_copy(dst_hbm.at[pl.ds(ibase + k * IROWS_DEG, IROWS_DEG)],
                        idx_v)

        def _body(r, carry2):
            pltpu.sync_copy(ones_v, acc_sh.at[idx_v.at[r]], add=True)
            return carry2

        lax.fori_loop(0, IROWS_DEG, _body, 0)
        return carry

    lax.fori_loop(0, ROWS_DEG // IROWS_DEG, _chunk, 0)
    plsc.subcore_barrier()

    def _wb(k, carry):
        off = s * NPT + k * 64
        pltpu.sync_copy(acc_sh.at[pl.ds(off, 64)], buf_v)
        pltpu.sync_copy(buf_v, out_hbm.at[c, pl.ds(off, 64)])
        return carry

    lax.fori_loop(0, NPT // 64, _wb, 0)


NCHUNK = 64
IROWS = 8
ROWS_C0 = 1280
TPT0 = ROWS_C0 // NUM_TILES
TPT1 = (EROWS - ROWS_C0) // NUM_TILES


@functools.partial(
    pl.kernel,
    mesh=_mesh,
    out_type=jax.ShapeDtypeStruct((NUM_SC, N_PAD, D), jnp.float32),
    scratch_types=[
        pltpu.VMEM((IROWS, 128), jnp.int32),
        pltpu.VMEM((IROWS, 128), jnp.int32),
        pltpu.VMEM((128, D), jnp.float32),
        pltpu.VMEM((128, D), jnp.float32),
        pltpu.VMEM((NCHUNK, D), jnp.float32),
        pltpu.VMEM_SHARED((N_PAD, D), jnp.float32),
        pltpu.SemaphoreType.DMA,
        pltpu.SemaphoreType.DMA,
    ],
)
def _edge_kernel(z_hbm, src_hbm, dst_hbm, out_hbm, srcv, dstv, rows_a, rows_b,
                 buf_v, acc_sh, sema, semb):
    c = lax.axis_index("c")
    s = lax.axis_index("s")

    @pl.when(c == 0)
    def _():
        def _init(k, carry):
            off = s * NPT + k * NCHUNK
            pltpu.sync_copy(z_hbm.at[pl.ds(off, NCHUNK)], buf_v)
            pltpu.sync_copy(buf_v, acc_sh.at[pl.ds(off, NCHUNK)])
            return carry

        lax.fori_loop(0, NPT // NCHUNK, _init, 0)

    @pl.when(c != 0)
    def _():
        def _zfill(r, carry):
            for j in range(D // 16):
                buf_v[r, pl.ds(j * 16, 16)] = jnp.zeros((16,), jnp.float32)
            return carry

        lax.fori_loop(0, NCHUNK, _zfill, 0)

        def _zinit(k, carry):
            pltpu.sync_copy(buf_v, acc_sh.at[pl.ds(s * NPT + k * NCHUNK,
                                                   NCHUNK)])
            return carry

        lax.fori_loop(0, NPT // NCHUNK, _zinit, 0)

    plsc.subcore_barrier()

    ibase = jnp.where(c == 0, s * TPT0, ROWS_C0 + s * TPT1)
    nchunks = jnp.where(c == 0, TPT0 // IROWS, TPT1 // IROWS)

    def _chunk(k, carry):
        pltpu.sync_copy(src_hbm.at[pl.ds(ibase + k * IROWS, IROWS)], srcv)
        pltpu.sync_copy(dst_hbm.at[pl.ds(ibase + k * IROWS, IROWS)], dstv)

        def _pair(j, carry2):
            r0 = 2 * j
            r1 = 2 * j + 1
            cpa = pltpu.async_copy(z_hbm.at[srcv.at[r0]], rows_a, sema)
            cpb = pltpu.async_copy(z_hbm.at[srcv.at[r1]], rows_b, semb)
            cpa.wait()
            pltpu.sync_copy(rows_a, acc_sh.at[dstv.at[r0]], add=True)
            cpb.wait()
            pltpu.sync_copy(rows_b, acc_sh.at[dstv.at[r1]], add=True)
            return carry2

        lax.fori_loop(0, IROWS // 2, _pair, 0)
        return carry

    lax.fori_loop(0, nchunks, _chunk, 0)
    plsc.subcore_barrier()

    def _wb(k, carry):
        off = s * NPT + k * NCHUNK
        pltpu.sync_copy(acc_sh.at[pl.ds(off, NCHUNK)], buf_v)
        pltpu.sync_copy(buf_v, out_hbm.at[c, pl.ds(off, NCHUNK)])
        return carry

    lax.fori_loop(0, NPT // NCHUNK, _wb, 0)


BR = 1024
GRID = N_PAD // BR


def _tc_prolog_body(deg_ref, x_ref, w_ref, z_ref, dinv_ref):
    p = deg_ref[...]
    deg = p[0, :, 0:1] + p[1, :, 0:1] + 1.0
    dinv = lax.rsqrt(deg)
    xw = jnp.dot(x_ref[...], w_ref[...], preferred_element_type=jnp.float32)
    z_ref[...] = dinv * xw
    dinv_ref[...] = dinv


_tc_prolog = pl.pallas_call(
    _tc_prolog_body,
    grid=(GRID,),
    in_specs=[
        pl.BlockSpec((NUM_SC, BR, 128), lambda i: (0, i, 0)),
        pl.BlockSpec((BR, D), lambda i: (i, 0)),
        pl.BlockSpec((D, D), lambda i: (0, 0)),
    ],
    out_specs=[
        pl.BlockSpec((BR, D), lambda i: (i, 0)),
        pl.BlockSpec((BR, 1), lambda i: (i, 0)),
    ],
    out_shape=[
        jax.ShapeDtypeStruct((N_PAD, D), jnp.float32),
        jax.ShapeDtypeStruct((N_PAD, 1), jnp.float32),
    ],
)


def _tc_mid_body(s_ref, dinv_ref, b_ref, w_ref, z2_ref):
    dinv = dinv_ref[...]
    agg = s_ref[0] + s_ref[1]
    h = jnp.maximum(dinv * agg + b_ref[...], 0.0)
    z2_ref[...] = dinv * jnp.dot(h, w_ref[...],
                                 preferred_element_type=jnp.float32)


_tc_mid = pl.pallas_call(
    _tc_mid_body,
    grid=(GRID,),
    in_specs=[
        pl.BlockSpec((NUM_SC, BR, D), lambda i: (0, i, 0)),
        pl.BlockSpec((BR, 1), lambda i: (i, 0)),
        pl.BlockSpec((1, D), lambda i: (0, 0)),
        pl.BlockSpec((D, D), lambda i: (0, 0)),
    ],
    out_specs=pl.BlockSpec((BR, D), lambda i: (i, 0)),
    out_shape=jax.ShapeDtypeStruct((N_PAD, D), jnp.float32),
)


def _tc_final_body(s_ref, dinv_ref, b_ref, out_ref):
    i = pl.program_id(0)
    dinv = dinv_ref[...]
    agg = s_ref[0] + s_ref[1]
    h = jnp.maximum(dinv * agg + b_ref[...], 0.0)
    rows = i * BR + lax.broadcasted_iota(jnp.int32, (BR, 1), 0)
    h = jnp.where(rows < N, h, 0.0)
    bmax = jnp.max(h, axis=0, keepdims=True)

    @pl.when(i == 0)
    def _():
        out_ref[...] = bmax

    @pl.when(i > 0)
    def _():
        out_ref[...] = jnp.maximum(out_ref[...], bmax)


_tc_final = pl.pallas_call(
    _tc_final_body,
    grid=(GRID,),
    in_specs=[
        pl.BlockSpec((NUM_SC, BR, D), lambda i: (0, i, 0)),
        pl.BlockSpec((BR, 1), lambda i: (i, 0)),
        pl.BlockSpec((1, D), lambda i: (0, 0)),
    ],
    out_specs=pl.BlockSpec((1, D), lambda i: (0, 0)),
    out_shape=jax.ShapeDtypeStruct((1, D), jnp.float32),
)


def kernel(x, edge_index, W1, b1, W2, b2):
    src = edge_index[0]
    dst = edge_index[1]
    pad_src = jnp.full((E_PAD - E,), N, jnp.int32)
    pad_dst = N + jnp.arange(E_PAD - E, dtype=jnp.int32) % (N_PAD - N)
    src2d = jnp.concatenate([src, pad_src]).reshape(EROWS, 128)
    dst2d = jnp.concatenate([dst, pad_dst]).reshape(EROWS, 128)
    x_pad = jnp.zeros((N_PAD, D), jnp.float32).at[:N].set(x)

    degp = _deg_kernel(dst2d, jnp.ones((128, 128), jnp.float32),
                       jnp.zeros((64, 128), jnp.float32))
    z1, dinv = _tc_prolog(degp, x_pad, W1)
    s1 = _edge_kernel(z1, src2d, dst2d)
    z2 = _tc_mid(s1, dinv, b1.reshape(1, D), W2)
    s2 = _edge_kernel(z2, src2d, dst2d)
    return _tc_final(s2, dinv, b2.reshape(1, D))


# speedup vs baseline: 1.1174x; 1.1174x over previous
import functools

import jax
import jax.numpy as jnp
from jax import lax
from jax.experimental import pallas as pl
from jax.experimental.pallas import tpu as pltpu
from jax.experimental.pallas import tpu_sc as plsc

N = 10000
D = 128
E = 320000
HALF = D // 2

NUM_SC = 2
NUM_TILES = 16
N_PAD = 10240
NPT = N_PAD // NUM_TILES
E_PAD = 327680
EROWS = E_PAD // 128
ROWS_MAIN = EROWS // NUM_TILES
ROWS_DEG = EROWS // (NUM_SC * NUM_TILES)

_mesh = plsc.VectorSubcoreMesh(core_axis_name="c", subcore_axis_name="s")


IROWS_DEG = 16


@functools.partial(
    pl.kernel,
    mesh=_mesh,
    out_type=jax.ShapeDtypeStruct((NUM_SC, N_PAD, 128), jnp.float32),
    scratch_types=[
        pltpu.VMEM((IROWS_DEG, 128), jnp.int32),
        pltpu.VMEM((128, 128), jnp.float32),
        pltpu.VMEM((64, 128), jnp.float32),
        pltpu.VMEM_SHARED((N_PAD, 128), jnp.float32),
    ],
)
def _deg_kernel(dst_hbm, ones_hbm, zeros_hbm, out_hbm, idx_v, ones_v, buf_v,
                acc_sh):
    c = lax.axis_index("c")
    s = lax.axis_index("s")
    wid = s * NUM_SC + c

    pltpu.sync_copy(ones_hbm, ones_v)
    pltpu.sync_copy(zeros_hbm, buf_v)

    def _zero(k, carry):
        pltpu.sync_copy(buf_v, acc_sh.at[pl.ds(s * NPT + k * 64, 64)])
        return carry

    lax.fori_loop(0, NPT // 64, _zero, 0)
    plsc.subcore_barrier()

    ibase = wid * ROWS_DEG

    def _chunk(k, carry):
        pltpu.sync_copy(dst_hbm.at[pl.ds(ibase + k * IROWS_DEG, IROWS_DEG)],
                        idx_v)

        def _body(r, carry2):
            pltpu.sync_copy(ones_v, acc_sh.at[idx_v.at[r]], add=True)
            return carry2

        lax.fori_loop(0, IROWS_DEG, _body, 0)
        return carry

    lax.fori_loop(0, ROWS_DEG // IROWS_DEG, _chunk, 0)
    plsc.subcore_barrier()

    def _wb(k, carry):
        off = s * NPT + k * 64
        pltpu.sync_copy(acc_sh.at[pl.ds(off, 64)], buf_v)
        pltpu.sync_copy(buf_v, out_hbm.at[c, pl.ds(off, 64)])
        return carry

    lax.fori_loop(0, NPT // 64, _wb, 0)


NCHUNK = 64
IROWS = 16
ROWS_C0 = 1792
TPT0 = ROWS_C0 // NUM_TILES
TPT1 = (EROWS - ROWS_C0) // NUM_TILES


@functools.partial(
    pl.kernel,
    mesh=_mesh,
    out_type=jax.ShapeDtypeStruct((NUM_SC, N_PAD, D), jnp.float32),
    scratch_types=[
        pltpu.VMEM((IROWS, 128), jnp.int32),
        pltpu.VMEM((IROWS, 128), jnp.int32),
        pltpu.VMEM((128, D), jnp.float32),
        pltpu.VMEM((128, D), jnp.float32),
        pltpu.VMEM((NCHUNK, D), jnp.float32),
        pltpu.VMEM_SHARED((N_PAD, D), jnp.float32),
        pltpu.SemaphoreType.DMA,
        pltpu.SemaphoreType.DMA,
    ],
)
def _edge_kernel(z_hbm, src_hbm, dst_hbm, out_hbm, srcv, dstv, rows_a, rows_b,
                 buf_v, acc_sh, sema, semb):
    c = lax.axis_index("c")
    s = lax.axis_index("s")

    @pl.when(c == 0)
    def _():
        def _init(k, carry):
            off = s * NPT + k * NCHUNK
            pltpu.sync_copy(z_hbm.at[pl.ds(off, NCHUNK)], buf_v)
            pltpu.sync_copy(buf_v, acc_sh.at[pl.ds(off, NCHUNK)])
            return carry

        lax.fori_loop(0, NPT // NCHUNK, _init, 0)

    @pl.when(c != 0)
    def _():
        def _zfill(r, carry):
            for j in range(D // 16):
                buf_v[r, pl.ds(j * 16, 16)] = jnp.zeros((16,), jnp.float32)
            return carry

        lax.fori_loop(0, NCHUNK, _zfill, 0)

        def _zinit(k, carry):
            pltpu.sync_copy(buf_v, acc_sh.at[pl.ds(s * NPT + k * NCHUNK,
                                                   NCHUNK)])
            return carry

        lax.fori_loop(0, NPT // NCHUNK, _zinit, 0)

    plsc.subcore_barrier()

    ibase = jnp.where(c == 0, s * TPT0, ROWS_C0 + s * TPT1)
    nchunks = jnp.where(c == 0, TPT0 // IROWS, TPT1 // IROWS)

    def _chunk(k, carry):
        pltpu.sync_copy(src_hbm.at[pl.ds(ibase + k * IROWS, IROWS)], srcv)
        pltpu.sync_copy(dst_hbm.at[pl.ds(ibase + k * IROWS, IROWS)], dstv)

        def _pair(j, carry2):
            r0 = 2 * j
            r1 = 2 * j + 1
            cpa = pltpu.async_copy(z_hbm.at[srcv.at[r0]], rows_a, sema)
            cpb = pltpu.async_copy(z_hbm.at[srcv.at[r1]], rows_b, semb)
            cpa.wait()
            pltpu.sync_copy(rows_a, acc_sh.at[dstv.at[r0]], add=True)
            cpb.wait()
            pltpu.sync_copy(rows_b, acc_sh.at[dstv.at[r1]], add=True)
            return carry2

        lax.fori_loop(0, IROWS // 2, _pair, 0)
        return carry

    lax.fori_loop(0, nchunks, _chunk, 0)
    plsc.subcore_barrier()

    def _wb(k, carry):
        off = s * NPT + k * NCHUNK
        pltpu.sync_copy(acc_sh.at[pl.ds(off, NCHUNK)], buf_v)
        pltpu.sync_copy(buf_v, out_hbm.at[c, pl.ds(off, NCHUNK)])
        return carry

    lax.fori_loop(0, NPT // NCHUNK, _wb, 0)


BR = 1024
GRID = N_PAD // BR


def _tc_prolog_body(deg_ref, x_ref, w_ref, z_ref, dinv_ref):
    p = deg_ref[...]
    deg = p[0, :, 0:1] + p[1, :, 0:1] + 1.0
    dinv = lax.rsqrt(deg)
    xw = jnp.dot(x_ref[...], w_ref[...], preferred_element_type=jnp.float32)
    z_ref[...] = dinv * xw
    dinv_ref[...] = dinv


_tc_prolog = pl.pallas_call(
    _tc_prolog_body,
    grid=(GRID,),
    in_specs=[
        pl.BlockSpec((NUM_SC, BR, 128), lambda i: (0, i, 0)),
        pl.BlockSpec((BR, D), lambda i: (i, 0)),
        pl.BlockSpec((D, D), lambda i: (0, 0)),
    ],
    out_specs=[
        pl.BlockSpec((BR, D), lambda i: (i, 0)),
        pl.BlockSpec((BR, 1), lambda i: (i, 0)),
    ],
    out_shape=[
        jax.ShapeDtypeStruct((N_PAD, D), jnp.float32),
        jax.ShapeDtypeStruct((N_PAD, 1), jnp.float32),
    ],
)


def _tc_mid_body(s_ref, dinv_ref, b_ref, w_ref, z2_ref):
    dinv = dinv_ref[...]
    agg = s_ref[0] + s_ref[1]
    h = jnp.maximum(dinv * agg + b_ref[...], 0.0)
    z2_ref[...] = dinv * jnp.dot(h, w_ref[...],
                                 preferred_element_type=jnp.float32)


_tc_mid = pl.pallas_call(
    _tc_mid_body,
    grid=(GRID,),
    in_specs=[
        pl.BlockSpec((NUM_SC, BR, D), lambda i: (0, i, 0)),
        pl.BlockSpec((BR, 1), lambda i: (i, 0)),
        pl.BlockSpec((1, D), lambda i: (0, 0)),
        pl.BlockSpec((D, D), lambda i: (0, 0)),
    ],
    out_specs=pl.BlockSpec((BR, D), lambda i: (i, 0)),
    out_shape=jax.ShapeDtypeStruct((N_PAD, D), jnp.float32),
)


def _tc_final_body(s_ref, dinv_ref, b_ref, out_ref):
    i = pl.program_id(0)
    dinv = dinv_ref[...]
    agg = s_ref[0] + s_ref[1]
    h = jnp.maximum(dinv * agg + b_ref[...], 0.0)
    rows = i * BR + lax.broadcasted_iota(jnp.int32, (BR, 1), 0)
    h = jnp.where(rows < N, h, 0.0)
    bmax = jnp.max(h, axis=0, keepdims=True)

    @pl.when(i == 0)
    def _():
        out_ref[...] = bmax

    @pl.when(i > 0)
    def _():
        out_ref[...] = jnp.maximum(out_ref[...], bmax)


_tc_final = pl.pallas_call(
    _tc_final_body,
    grid=(GRID,),
    in_specs=[
        pl.BlockSpec((NUM_SC, BR, D), lambda i: (0, i, 0)),
        pl.BlockSpec((BR, 1), lambda i: (i, 0)),
        pl.BlockSpec((1, D), lambda i: (0, 0)),
    ],
    out_specs=pl.BlockSpec((1, D), lambda i: (0, 0)),
    out_shape=jax.ShapeDtypeStruct((1, D), jnp.float32),
)


def kernel(x, edge_index, W1, b1, W2, b2):
    src = edge_index[0]
    dst = edge_index[1]
    pad_src = jnp.full((E_PAD - E,), N, jnp.int32)
    pad_dst = N + jnp.arange(E_PAD - E, dtype=jnp.int32) % (N_PAD - N)
    src2d = jnp.concatenate([src, pad_src]).reshape(EROWS, 128)
    dst2d = jnp.concatenate([dst, pad_dst]).reshape(EROWS, 128)
    x_pad = jnp.zeros((N_PAD, D), jnp.float32).at[:N].set(x)

    degp = _deg_kernel(dst2d, jnp.ones((128, 128), jnp.float32),
                       jnp.zeros((64, 128), jnp.float32))
    z1, dinv = _tc_prolog(degp, x_pad, W1)
    s1 = _edge_kernel(z1, src2d, dst2d)
    z2 = _tc_mid(s1, dinv, b1.reshape(1, D), W2)
    s2 = _edge_kernel(z2, src2d, dst2d)
    return _tc_final(s2, dinv, b2.reshape(1, D))


# speedup vs baseline: 1.1552x; 1.0338x over previous
import functools

import jax
import jax.numpy as jnp
from jax import lax
from jax.experimental import pallas as pl
from jax.experimental.pallas import tpu as pltpu
from jax.experimental.pallas import tpu_sc as plsc

N = 10000
D = 128
E = 320000
HALF = D // 2

NUM_SC = 2
NUM_TILES = 16
N_PAD = 10240
NPT = N_PAD // NUM_TILES
E_PAD = 327680
EROWS = E_PAD // 128
ROWS_MAIN = EROWS // NUM_TILES
ROWS_DEG = EROWS // (NUM_SC * NUM_TILES)

_mesh = plsc.VectorSubcoreMesh(core_axis_name="c", subcore_axis_name="s")


IROWS_DEG = 16


@functools.partial(
    pl.kernel,
    mesh=_mesh,
    out_type=jax.ShapeDtypeStruct((NUM_SC, N_PAD, 128), jnp.float32),
    scratch_types=[
        pltpu.VMEM((IROWS_DEG, 128), jnp.int32),
        pltpu.VMEM((128, 128), jnp.float32),
        pltpu.VMEM((64, 128), jnp.float32),
        pltpu.VMEM_SHARED((N_PAD, 128), jnp.float32),
    ],
)
def _deg_kernel(dst_hbm, ones_hbm, zeros_hbm, out_hbm, idx_v, ones_v, buf_v,
                acc_sh):
    c = lax.axis_index("c")
    s = lax.axis_index("s")
    wid = s * NUM_SC + c

    pltpu.sync_copy(ones_hbm, ones_v)
    pltpu.sync_copy(zeros_hbm, buf_v)

    def _zero(k, carry):
        pltpu.sync_copy(buf_v, acc_sh.at[pl.ds(s * NPT + k * 64, 64)])
        return carry

    lax.fori_loop(0, NPT // 64, _zero, 0)
    plsc.subcore_barrier()

    ibase = wid * ROWS_DEG

    def _chunk(k, carry):
        pltpu.sync_copy(dst_hbm.at[pl.ds(ibase + k * IROWS_DEG, IROWS_DEG)],
                        idx_v)

        def _body(r, carry2):
            pltpu.sync_copy(ones_v, acc_sh.at[idx_v.at[r]], add=True)
            return carry2

        lax.fori_loop(0, IROWS_DEG, _body, 0)
        return carry

    lax.fori_loop(0, ROWS_DEG // IROWS_DEG, _chunk, 0)
    plsc.subcore_barrier()

    def _wb(k, carry):
        off = s * NPT + k * 64
        pltpu.sync_copy(acc_sh.at[pl.ds(off, 64)], buf_v)
        pltpu.sync_copy(buf_v, out_hbm.at[c, pl.ds(off, 64)])
        return carry

    lax.fori_loop(0, NPT // 64, _wb, 0)


NCHUNK = 64
IROWS = 8
ROWS_C0 = 1920
TPT0 = ROWS_C0 // NUM_TILES
TPT1 = (EROWS - ROWS_C0) // NUM_TILES


@functools.partial(
    pl.kernel,
    mesh=_mesh,
    out_type=jax.ShapeDtypeStruct((NUM_SC, N_PAD, D), jnp.float32),
    scratch_types=[
        pltpu.VMEM((IROWS, 128), jnp.int32),
        pltpu.VMEM((IROWS, 128), jnp.int32),
        pltpu.VMEM((128, D), jnp.float32),
        pltpu.VMEM((128, D), jnp.float32),
        pltpu.VMEM((NCHUNK, D), jnp.float32),
        pltpu.VMEM_SHARED((N_PAD, D), jnp.float32),
        pltpu.SemaphoreType.DMA,
        pltpu.SemaphoreType.DMA,
    ],
)
def _edge_kernel(z_hbm, src_hbm, dst_hbm, out_hbm, srcv, dstv, rows_a, rows_b,
                 buf_v, acc_sh, sema, semb):
    c = lax.axis_index("c")
    s = lax.axis_index("s")

    @pl.when(c == 0)
    def _():
        def _init(k, carry):
            off = s * NPT + k * NCHUNK
            pltpu.sync_copy(z_hbm.at[pl.ds(off, NCHUNK)], buf_v)
            pltpu.sync_copy(buf_v, acc_sh.at[pl.ds(off, NCHUNK)])
            return carry

        lax.fori_loop(0, NPT // NCHUNK, _init, 0)

    @pl.when(c != 0)
    def _():
        def _zfill(r, carry):
            for j in range(D // 16):
                buf_v[r, pl.ds(j * 16, 16)] = jnp.zeros((16,), jnp.float32)
            return carry

        lax.fori_loop(0, NCHUNK, _zfill, 0)

        def _zinit(k, carry):
            pltpu.sync_copy(buf_v, acc_sh.at[pl.ds(s * NPT + k * NCHUNK,
                                                   NCHUNK)])
            return carry

        lax.fori_loop(0, NPT // NCHUNK, _zinit, 0)

    plsc.subcore_barrier()

    ibase = jnp.where(c == 0, s * TPT0, ROWS_C0 + s * TPT1)
    nchunks = jnp.where(c == 0, TPT0 // IROWS, TPT1 // IROWS)

    def _chunk(k, carry):
        pltpu.sync_copy(src_hbm.at[pl.ds(ibase + k * IROWS, IROWS)], srcv)
        pltpu.sync_copy(dst_hbm.at[pl.ds(ibase + k * IROWS, IROWS)], dstv)

        def _pair(j, carry2):
            r0 = 2 * j
            r1 = 2 * j + 1
            cpa = pltpu.async_copy(z_hbm.at[srcv.at[r0]], rows_a, sema)
            cpb = pltpu.async_copy(z_hbm.at[srcv.at[r1]], rows_b, semb)
            cpa.wait()
            pltpu.sync_copy(rows_a, acc_sh.at[dstv.at[r0]], add=True)
            cpb.wait()
            pltpu.sync_copy(rows_b, acc_sh.at[dstv.at[r1]], add=True)
            return carry2

        lax.fori_loop(0, IROWS // 2, _pair, 0)
        return carry

    lax.fori_loop(0, nchunks, _chunk, 0)
    plsc.subcore_barrier()

    def _wb(k, carry):
        off = s * NPT + k * NCHUNK
        pltpu.sync_copy(acc_sh.at[pl.ds(off, NCHUNK)], buf_v)
        pltpu.sync_copy(buf_v, out_hbm.at[c, pl.ds(off, NCHUNK)])
        return carry

    lax.fori_loop(0, NPT // NCHUNK, _wb, 0)


BR = 1024
GRID = N_PAD // BR


def _tc_prolog_body(deg_ref, x_ref, w_ref, z_ref, dinv_ref):
    p = deg_ref[...]
    deg = p[0, :, 0:1] + p[1, :, 0:1] + 1.0
    dinv = lax.rsqrt(deg)
    xw = jnp.dot(x_ref[...], w_ref[...], preferred_element_type=jnp.float32)
    z_ref[...] = dinv * xw
    dinv_ref[...] = dinv


_tc_prolog = pl.pallas_call(
    _tc_prolog_body,
    grid=(GRID,),
    in_specs=[
        pl.BlockSpec((NUM_SC, BR, 128), lambda i: (0, i, 0)),
        pl.BlockSpec((BR, D), lambda i: (i, 0)),
        pl.BlockSpec((D, D), lambda i: (0, 0)),
    ],
    out_specs=[
        pl.BlockSpec((BR, D), lambda i: (i, 0)),
        pl.BlockSpec((BR, 1), lambda i: (i, 0)),
    ],
    out_shape=[
        jax.ShapeDtypeStruct((N_PAD, D), jnp.float32),
        jax.ShapeDtypeStruct((N_PAD, 1), jnp.float32),
    ],
)


def _tc_mid_body(s_ref, dinv_ref, b_ref, w_ref, z2_ref):
    dinv = dinv_ref[...]
    agg = s_ref[0] + s_ref[1]
    h = jnp.maximum(dinv * agg + b_ref[...], 0.0)
    z2_ref[...] = dinv * jnp.dot(h, w_ref[...],
                                 preferred_element_type=jnp.float32)


_tc_mid = pl.pallas_call(
    _tc_mid_body,
    grid=(GRID,),
    in_specs=[
        pl.BlockSpec((NUM_SC, BR, D), lambda i: (0, i, 0)),
        pl.BlockSpec((BR, 1), lambda i: (i, 0)),
        pl.BlockSpec((1, D), lambda i: (0, 0)),
        pl.BlockSpec((D, D), lambda i: (0, 0)),
    ],
    out_specs=pl.BlockSpec((BR, D), lambda i: (i, 0)),
    out_shape=jax.ShapeDtypeStruct((N_PAD, D), jnp.float32),
)


def _tc_final_body(s_ref, dinv_ref, b_ref, out_ref):
    i = pl.program_id(0)
    dinv = dinv_ref[...]
    agg = s_ref[0] + s_ref[1]
    h = jnp.maximum(dinv * agg + b_ref[...], 0.0)
    rows = i * BR + lax.broadcasted_iota(jnp.int32, (BR, 1), 0)
    h = jnp.where(rows < N, h, 0.0)
    bmax = jnp.max(h, axis=0, keepdims=True)

    @pl.when(i == 0)
    def _():
        out_ref[...] = bmax

    @pl.when(i > 0)
    def _():
        out_ref[...] = jnp.maximum(out_ref[...], bmax)


_tc_final = pl.pallas_call(
    _tc_final_body,
    grid=(GRID,),
    in_specs=[
        pl.BlockSpec((NUM_SC, BR, D), lambda i: (0, i, 0)),
        pl.BlockSpec((BR, 1), lambda i: (i, 0)),
        pl.BlockSpec((1, D), lambda i: (0, 0)),
    ],
    out_specs=pl.BlockSpec((1, D), lambda i: (0, 0)),
    out_shape=jax.ShapeDtypeStruct((1, D), jnp.float32),
)


def kernel(x, edge_index, W1, b1, W2, b2):
    src = edge_index[0]
    dst = edge_index[1]
    pad_src = jnp.full((E_PAD - E,), N, jnp.int32)
    pad_dst = N + jnp.arange(E_PAD - E, dtype=jnp.int32) % (N_PAD - N)
    src2d = jnp.concatenate([src, pad_src]).reshape(EROWS, 128)
    dst2d = jnp.concatenate([dst, pad_dst]).reshape(EROWS, 128)
    x_pad = jnp.zeros((N_PAD, D), jnp.float32).at[:N].set(x)

    degp = _deg_kernel(dst2d, jnp.ones((128, 128), jnp.float32),
                       jnp.zeros((64, 128), jnp.float32))
    z1, dinv = _tc_prolog(degp, x_pad, W1)
    s1 = _edge_kernel(z1, src2d, dst2d)
    z2 = _tc_mid(s1, dinv, b1.reshape(1, D), W2)
    s2 = _edge_kernel(z2, src2d, dst2d)
    return _tc_final(s2, dinv, b2.reshape(1, D))
